# baseline (speedup 1.0000x reference)
import dataclasses
import functools

import jax
import jax.numpy as jnp
from jax import lax
from jax.experimental import pallas as pl
from jax.experimental.pallas import tpu as pltpu
from jax.experimental.pallas import tpu_sc as plsc

_N = 10000
_E = 320000
_D = 128
_G = 64
_C = 8

_NC = 2
_NS = 16
_L = 16

_K = 128
_KG = 64
_SLOW = 0
_CHS = 80
_CHF = 80
_RINGG = 4
_ES = _NS * _CHS * _K
_EF = _NS * _CHF * _K
_NPAD = 10112
_RPS = _NPAD // _NS
_HLF = 2
_CHHX = _CHF // _HLF

_NBLK = 10
_BR = _N // _NBLK

_RING = 2


def _sc_mesh():
    return plsc.VectorSubcoreMesh(
        core_axis_name="c", subcore_axis_name="s",
        num_cores=_NC, num_subcores=_NS)


_HR = 80


def _sc_degree(dst3, iota_rows, zero_rows):
    @functools.partial(
        pl.kernel,
        out_type=jax.ShapeDtypeStruct((_NC * _HR, _K), jnp.float32),
        mesh=_sc_mesh(),
        compiler_params=dataclasses.replace(pltpu.CompilerParams(),
                                            needs_layout_passes=False),
        scratch_types=[
            pltpu.VMEM((_HLF * _CHHX, _K), jnp.int32),
            pltpu.VMEM((_HR, _K), jnp.float32),
            pltpu.VMEM((1, _HR), jnp.int32),
            pltpu.VMEM_SHARED((_HR, _K), jnp.float32),
            pltpu.SemaphoreType.DMA,
        ],
    )
    def deg_kernel(dst_hbm, iota_hbm, zero_hbm, out_hbm,
                   dstv, histv, iotav, accsh, sem):
        c = lax.axis_index("c")
        s = lax.axis_index("s")
        w = c * _NS + s
        pltpu.sync_copy(zero_hbm, histv)
        pltpu.sync_copy(iota_hbm, iotav)
        for half in range(_HLF):
            pltpu.sync_copy(dst_hbm.at[w * _HLF + half],
                            dstv.at[pl.ds(half * _CHHX, _CHHX)])

        @pl.when(s < _HR // 8)
        def _():
            pltpu.sync_copy(zero_hbm.at[pl.ds(s * 8, 8)],
                            accsh.at[pl.ds(s * 8, 8)])

        ones = jnp.ones((_L,), jnp.float32)

        @pl.loop(0, _HLF * _CHHX)
        def _(j):
            @pl.loop(0, _K // _L)
            def _(t):
                idx = dstv[j, pl.ds(t * _L, _L)]
                plsc.addupdate_scatter(histv, [idx >> 7, idx & 127], ones)

        plsc.subcore_barrier()
        pltpu.sync_copy(histv, accsh.at[iotav.at[0]], add=True)
        plsc.subcore_barrier()

        @pl.when(s < _HR // 8)
        def _():
            pltpu.sync_copy(accsh.at[pl.ds(s * 8, 8)],
                            out_hbm.at[pl.ds(c * _HR + s * 8, 8)])

    return deg_kernel(dst3, iota_rows, zero_rows)


def _sc_aggregate(h, src3, dst3, zero_rows):
    @functools.partial(
        pl.kernel,
        out_type=jax.ShapeDtypeStruct((_NC * _NPAD, _D), jnp.float32),
        mesh=_sc_mesh(),
        scratch_types=[
            pltpu.VMEM((_CHHX, _K), jnp.int32),
            pltpu.VMEM((_CHHX, _K), jnp.int32),
            pltpu.VMEM((_K, _D), jnp.float32),
            pltpu.VMEM((_K, _D), jnp.float32),
            pltpu.SemaphoreType.DMA,
            pltpu.SemaphoreType.DMA,
            pltpu.SemaphoreType.DMA,
            pltpu.SemaphoreType.DMA,
            pltpu.VMEM_SHARED((_NPAD, _D), jnp.float32),
        ],
    )
    def agg_kernel(h_hbm, src_hbm, dst_hbm, zero_hbm, out_hbm,
                   srcv, dstv, rb0, rb1, sm0, sm1, ss0, ss1, accsh):
        rowbufs = (rb0, rb1)
        gsems = (sm0, sm1)
        ssems = (ss0, ss1)
        c = lax.axis_index("c")
        s = lax.axis_index("s")
        w = c * _NS + s
        pltpu.sync_copy(zero_hbm, accsh.at[pl.ds(s * _RPS, _RPS)])
        plsc.subcore_barrier()

        def wait_g(r):
            pltpu.make_async_copy(h_hbm.at[srcv.at[0]], rowbufs[r],
                                  gsems[r]).wait()

        def wait_s(r):
            pltpu.make_async_copy(rowbufs[r], accsh.at[dstv.at[0]],
                                  ssems[r]).wait()

        def run_edges(n_half):
            for half in range(_HLF):
                pltpu.sync_copy(src_hbm.at[w * _HLF + half], srcv)
                pltpu.sync_copy(dst_hbm.at[w * _HLF + half], dstv)
                for r in range(_RING):
                    pltpu.async_copy(h_hbm.at[srcv.at[r]], rowbufs[r],
                                     gsems[r])

                @pl.loop(0, n_half - _RING, step=_RING)
                def _(j):
                    for r in range(_RING):
                        wait_g(r)
                        pltpu.async_copy(rowbufs[r], accsh.at[dstv.at[j + r]],
                                         ssems[r], add=True)
                    for r in range(_RING):
                        wait_s(r)
                        pltpu.async_copy(h_hbm.at[srcv.at[j + _RING + r]],
                                         rowbufs[r], gsems[r])

                for r in range(_RING):
                    wait_g(r)
                    pltpu.async_copy(rowbufs[r],
                                     accsh.at[dstv.at[n_half - _RING + r]],
                                     ssems[r], add=True)
                for r in range(_RING):
                    wait_s(r)

        run_edges(_CHS // _HLF)

        plsc.subcore_barrier()
        pltpu.sync_copy(accsh.at[pl.ds(s * _RPS, _RPS)],
                        out_hbm.at[pl.ds(c * _NPAD + s * _RPS, _RPS)])

    return agg_kernel(h, src3, dst3, zero_rows)


def _dis_from(degA, degB):
    deg = degA[0, :, :1] + degB[0, :, :1] + 1.0
    return lax.rsqrt(deg)


def _tc_scale_matmul(x, W1, deg2):
    def body(x_ref, w_ref, dA_ref, dB_ref, o_ref):
        dis = _dis_from(dA_ref[...], dB_ref[...])
        h = jnp.dot(x_ref[...], w_ref[...], preferred_element_type=jnp.float32)
        o_ref[...] = h * dis

    return pl.pallas_call(
        body,
        grid=(_NBLK,),
        in_specs=[
            pl.BlockSpec((_BR, _D), lambda i: (i, 0)),
            pl.BlockSpec((_D, _D), lambda i: (0, 0)),
            pl.BlockSpec((1, _BR, 1), lambda i: (0, i, 0)),
            pl.BlockSpec((1, _BR, 1), lambda i: (1, i, 0)),
        ],
        out_specs=pl.BlockSpec((_BR, _D), lambda i: (i, 0)),
        out_shape=jax.ShapeDtypeStruct((_N, _D), jnp.float32),
    )(x, W1, deg2, deg2)


def _tc_mid_layer(acc1, h1p, deg2, b1, W2):
    def body(aA_ref, aB_ref, h_ref, dA_ref, dB_ref, b_ref, w_ref, o_ref):
        dis = _dis_from(dA_ref[...], dB_ref[...])
        pre = (aA_ref[0] + aB_ref[0] + h_ref[...]) * dis + b_ref[...]
        x2 = jnp.maximum(pre, 0.0)
        o_ref[...] = jnp.dot(x2, w_ref[...],
                             preferred_element_type=jnp.float32) * dis

    return pl.pallas_call(
        body,
        grid=(_NBLK,),
        in_specs=[
            pl.BlockSpec((1, _BR, _D), lambda i: (0, i, 0)),
            pl.BlockSpec((1, _BR, _D), lambda i: (1, i, 0)),
            pl.BlockSpec((_BR, _D), lambda i: (i, 0)),
            pl.BlockSpec((1, _BR, 1), lambda i: (0, i, 0)),
            pl.BlockSpec((1, _BR, 1), lambda i: (1, i, 0)),
            pl.BlockSpec((1, _D), lambda i: (0, 0)),
            pl.BlockSpec((_D, _D), lambda i: (0, 0)),
        ],
        out_specs=pl.BlockSpec((_BR, _D), lambda i: (i, 0)),
        out_shape=jax.ShapeDtypeStruct((_N, _D), jnp.float32),
    )(acc1, acc1, h1p, deg2, deg2, b1, W2)


def _tc_post_layer(acc2, h2p, deg2, b2):
    def body(aA_ref, aB_ref, h_ref, dA_ref, dB_ref, b_ref, o_ref):
        dis = _dis_from(dA_ref[...], dB_ref[...])
        pre = (aA_ref[0] + aB_ref[0] + h_ref[...]) * dis + b_ref[...]
        o_ref[...] = jnp.maximum(pre, 0.0)

    return pl.pallas_call(
        body,
        grid=(_NBLK,),
        in_specs=[
            pl.BlockSpec((1, _BR, _D), lambda i: (0, i, 0)),
            pl.BlockSpec((1, _BR, _D), lambda i: (1, i, 0)),
            pl.BlockSpec((_BR, _D), lambda i: (i, 0)),
            pl.BlockSpec((1, _BR, 1), lambda i: (0, i, 0)),
            pl.BlockSpec((1, _BR, 1), lambda i: (1, i, 0)),
            pl.BlockSpec((1, _D), lambda i: (0, 0)),
        ],
        out_specs=pl.BlockSpec((_BR, _D), lambda i: (i, 0)),
        out_shape=jax.ShapeDtypeStruct((_N, _D), jnp.float32),
    )(acc2, acc2, h2p, deg2, deg2, b2)


def _tc_head(h2, batch_col, f1W, f1b, f2W, f2b):
    def body(h_ref, b_ref, w1_ref, b1_ref, w2_ref, b2_ref, o_ref, gf_ref):
        h3 = h_ref[...].reshape(_N // 8, 8, _D)
        b3 = b_ref[...].reshape(_N // 8, 8, 1)
        ids0 = b3[:, 0, :]
        ids7 = b3[:, 7, :]
        low = jnp.max(jnp.where(b3 == ids0[:, None, :], h3, -jnp.inf), axis=1)
        high = jnp.max(jnp.where(b3 == ids7[:, None, :], h3, -jnp.inf), axis=1)

        def seg(g, carry):
            ml = jnp.max(jnp.where(ids0 == g, low, -jnp.inf),
                         axis=0, keepdims=True)
            mh = jnp.max(jnp.where(ids7 == g, high, -jnp.inf),
                         axis=0, keepdims=True)
            gf_ref[pl.ds(g, 1), :] = jnp.maximum(ml, mh)
            return carry

        lax.fori_loop(0, _G, seg, 0)

        @pl.when(jnp.max(ids7 - ids0) >= 2)
        def _():
            def seg_full(g, carry):
                m = jnp.where(b_ref[...] == g, h_ref[...], -jnp.inf)
                gf_ref[pl.ds(g, 1), :] = jnp.max(m, axis=0, keepdims=True)
                return carry

            lax.fori_loop(0, _G, seg_full, 0)
        gf = gf_ref[...]
        z1 = jnp.dot(gf, w1_ref[...], preferred_element_type=jnp.float32)
        z1 = jnp.maximum(z1 + b1_ref[...], 0.0)
        z2 = jnp.dot(z1, w2_ref[...], preferred_element_type=jnp.float32)
        z2 = z2 + b2_ref[...]
        zmax = jnp.max(z2, axis=1, keepdims=True)
        lse = jnp.log(jnp.sum(jnp.exp(z2 - zmax), axis=1, keepdims=True)) + zmax
        o_ref[...] = z2 - lse

    return pl.pallas_call(
        body,
        scratch_shapes=[pltpu.VMEM((_G, _D), jnp.float32)],
        out_shape=jax.ShapeDtypeStruct((_G, _D), jnp.float32),
    )(h2, batch_col, f1W, f1b, f2W, f2b)


def kernel(x, edge_index, batch, W1, b1, W2, b2, fc1_W, fc1_b, fc2_W, fc2_b):
    src = edge_index[0].astype(jnp.int32)
    dst = edge_index[1].astype(jnp.int32)
    npad = _ES + _EF - _E
    src3 = jnp.concatenate(
        [src, jnp.zeros((npad,), jnp.int32)]).reshape(
            _NC * _NS * _HLF, _CHHX, _K)
    dst3 = jnp.concatenate(
        [dst, jnp.full((npad,), _N, jnp.int32)]).reshape(
            _NC * _NS * _HLF, _CHHX, _K)

    zero_hist = jnp.zeros((_HR, _K), jnp.float32)
    zero_rows = jnp.zeros((_RPS, _D), jnp.float32)
    iota_rows = jnp.arange(_HR, dtype=jnp.int32).reshape(1, _HR)

    deg2 = _sc_degree(dst3, iota_rows, zero_hist).reshape(_NC, _HR * _K, 1)

    h1p = _tc_scale_matmul(x, W1, deg2)
    acc1 = _sc_aggregate(h1p, src3, dst3, zero_rows).reshape(_NC, _NPAD, _D)
    h2p = _tc_mid_layer(acc1, h1p, deg2, b1.reshape(1, _D), W2)
    acc2 = _sc_aggregate(h2p, src3, dst3, zero_rows).reshape(_NC, _NPAD, _D)
    h2 = _tc_post_layer(acc2, h2p, deg2, b2.reshape(1, _D))

    f1W = jnp.pad(fc1_W, ((0, 0), (0, _D - fc1_W.shape[1])))
    f1b = jnp.pad(fc1_b, (0, _D - fc1_b.shape[0])).reshape(1, _D)
    f2W = jnp.pad(fc2_W, ((0, _D - fc2_W.shape[0]), (0, _D - fc2_W.shape[1])))
    f2b = jnp.pad(fc2_b, (0, _D - fc2_b.shape[0]),
                  constant_values=-1e30).reshape(1, _D)

    out = _tc_head(h2, batch.astype(jnp.int32).reshape(_N, 1),
                   f1W, f1b, f2W, f2b)
    return out[:, :_C]


# speedup vs baseline: 1.0050x; 1.0050x over previous
import dataclasses
import functools

import jax
import jax.numpy as jnp
from jax import lax
from jax.experimental import pallas as pl
from jax.experimental.pallas import tpu as pltpu
from jax.experimental.pallas import tpu_sc as plsc

_N = 10000
_E = 320000
_D = 128
_G = 64
_C = 8

_NC = 2
_NS = 16
_L = 16

_K = 128
_KG = 64
_SLOW = 0
_CHS = 80
_CHF = 80
_RINGG = 4
_ES = _NS * _CHS * _K
_EF = _NS * _CHF * _K
_NPAD = 10112
_RPS = _NPAD // _NS
_HLF = 2
_CHHX = _CHF // _HLF

_NBLK = 10
_BR = _N // _NBLK

_RING = 2


def _sc_mesh():
    return plsc.VectorSubcoreMesh(
        core_axis_name="c", subcore_axis_name="s",
        num_cores=_NC, num_subcores=_NS)


_HR = 80


def _sc_degree(dst3, iota_rows, zero_rows):
    @functools.partial(
        pl.kernel,
        out_type=jax.ShapeDtypeStruct((_NC * _HR, _K), jnp.float32),
        mesh=_sc_mesh(),
        compiler_params=dataclasses.replace(pltpu.CompilerParams(),
                                            needs_layout_passes=False),
        scratch_types=[
            pltpu.VMEM((_HLF * _CHHX, _K), jnp.int32),
            pltpu.VMEM((_HR, _K), jnp.float32),
            pltpu.VMEM((1, _HR), jnp.int32),
            pltpu.VMEM_SHARED((_HR, _K), jnp.float32),
            pltpu.SemaphoreType.DMA,
        ],
    )
    def deg_kernel(dst_hbm, iota_hbm, zero_hbm, out_hbm,
                   dstv, histv, iotav, accsh, sem):
        c = lax.axis_index("c")
        s = lax.axis_index("s")
        w = c * _NS + s
        pltpu.sync_copy(zero_hbm, histv)
        pltpu.sync_copy(iota_hbm, iotav)
        for half in range(_HLF):
            pltpu.sync_copy(dst_hbm.at[w * _HLF + half],
                            dstv.at[pl.ds(half * _CHHX, _CHHX)])

        @pl.when(s < _HR // 8)
        def _():
            pltpu.sync_copy(zero_hbm.at[pl.ds(s * 8, 8)],
                            accsh.at[pl.ds(s * 8, 8)])

        ones = jnp.ones((_L,), jnp.float32)

        @pl.loop(0, _HLF * _CHHX)
        def _(j):
            @pl.loop(0, _K // _L)
            def _(t):
                idx = dstv[j, pl.ds(t * _L, _L)]
                plsc.addupdate_scatter(histv, [idx >> 7, idx & 127], ones)

        plsc.subcore_barrier()
        pltpu.sync_copy(histv, accsh.at[iotav.at[0]], add=True)
        plsc.subcore_barrier()

        @pl.when(s < _HR // 8)
        def _():
            pltpu.sync_copy(accsh.at[pl.ds(s * 8, 8)],
                            out_hbm.at[pl.ds(c * _HR + s * 8, 8)])

    return deg_kernel(dst3, iota_rows, zero_rows)


def _sc_aggregate(h, src3, dst3, zero_rows):
    @functools.partial(
        pl.kernel,
        out_type=jax.ShapeDtypeStruct((_NC * _NPAD, _D), jnp.float32),
        mesh=_sc_mesh(),
        scratch_types=[
            pltpu.VMEM((_CHHX, _K), jnp.int32),
            pltpu.VMEM((_CHHX, _K), jnp.int32),
            pltpu.VMEM((_K, _D), jnp.float32),
            pltpu.VMEM((_K, _D), jnp.float32),
            pltpu.SemaphoreType.DMA,
            pltpu.SemaphoreType.DMA,
            pltpu.SemaphoreType.DMA,
            pltpu.SemaphoreType.DMA,
            pltpu.VMEM_SHARED((_NPAD, _D), jnp.float32),
        ],
    )
    def agg_kernel(h_hbm, src_hbm, dst_hbm, zero_hbm, out_hbm,
                   srcv, dstv, rb0, rb1, sm0, sm1, ss0, ss1, accsh):
        rowbufs = (rb0, rb1)
        gsems = (sm0, sm1)
        ssems = (ss0, ss1)
        c = lax.axis_index("c")
        s = lax.axis_index("s")
        w = c * _NS + s
        pltpu.sync_copy(zero_hbm, accsh.at[pl.ds(s * _RPS, _RPS)])
        plsc.subcore_barrier()

        def wait_g(r):
            pltpu.make_async_copy(h_hbm.at[srcv.at[0]], rowbufs[r],
                                  gsems[r]).wait()

        def wait_s(r):
            pltpu.make_async_copy(rowbufs[r], accsh.at[dstv.at[0]],
                                  ssems[r]).wait()

        def run_edges(n_half):
            for half in range(_HLF):
                pltpu.sync_copy(src_hbm.at[w * _HLF + half], srcv)
                pltpu.sync_copy(dst_hbm.at[w * _HLF + half], dstv)
                for r in range(_RING):
                    pltpu.async_copy(h_hbm.at[srcv.at[r]], rowbufs[r],
                                     gsems[r])

                @pl.loop(0, n_half - _RING, step=_RING)
                def _(j):
                    for r in range(_RING):
                        wait_g(r)
                        pltpu.async_copy(rowbufs[r], accsh.at[dstv.at[j + r]],
                                         ssems[r], add=True)
                    for r in range(_RING):
                        wait_s(r)
                        pltpu.async_copy(h_hbm.at[srcv.at[j + _RING + r]],
                                         rowbufs[r], gsems[r])

                for r in range(_RING):
                    wait_g(r)
                    pltpu.async_copy(rowbufs[r],
                                     accsh.at[dstv.at[n_half - _RING + r]],
                                     ssems[r], add=True)
                for r in range(_RING):
                    wait_s(r)

        run_edges(_CHS // _HLF)

        plsc.subcore_barrier()
        pltpu.sync_copy(accsh.at[pl.ds(s * _RPS, _RPS)],
                        out_hbm.at[pl.ds(c * _NPAD + s * _RPS, _RPS)])

    return agg_kernel(h, src3, dst3, zero_rows)


def _dis_from(degA, degB):
    deg = degA[0, :, :1] + degB[0, :, :1] + 1.0
    return lax.rsqrt(deg)


def _tc_scale_matmul(x, W1, deg2):
    def body(x_ref, w_ref, dA_ref, dB_ref, o_ref):
        dis = _dis_from(dA_ref[...], dB_ref[...])
        h = jnp.dot(x_ref[...], w_ref[...], preferred_element_type=jnp.float32)
        o_ref[...] = h * dis

    return pl.pallas_call(
        body,
        grid=(_NBLK,),
        in_specs=[
            pl.BlockSpec((_BR, _D), lambda i: (i, 0)),
            pl.BlockSpec((_D, _D), lambda i: (0, 0)),
            pl.BlockSpec((1, _BR, 1), lambda i: (0, i, 0)),
            pl.BlockSpec((1, _BR, 1), lambda i: (1, i, 0)),
        ],
        out_specs=pl.BlockSpec((_BR, _D), lambda i: (i, 0)),
        out_shape=jax.ShapeDtypeStruct((_N, _D), jnp.float32),
    )(x, W1, deg2, deg2)


def _tc_mid_layer(acc1, h1p, deg2, b1, W2):
    def body(aA_ref, aB_ref, h_ref, dA_ref, dB_ref, b_ref, w_ref, o_ref):
        dis = _dis_from(dA_ref[...], dB_ref[...])
        pre = (aA_ref[0] + aB_ref[0] + h_ref[...]) * dis + b_ref[...]
        x2 = jnp.maximum(pre, 0.0)
        o_ref[...] = jnp.dot(x2, w_ref[...],
                             preferred_element_type=jnp.float32) * dis

    return pl.pallas_call(
        body,
        grid=(_NBLK,),
        in_specs=[
            pl.BlockSpec((1, _BR, _D), lambda i: (0, i, 0)),
            pl.BlockSpec((1, _BR, _D), lambda i: (1, i, 0)),
            pl.BlockSpec((_BR, _D), lambda i: (i, 0)),
            pl.BlockSpec((1, _BR, 1), lambda i: (0, i, 0)),
            pl.BlockSpec((1, _BR, 1), lambda i: (1, i, 0)),
            pl.BlockSpec((1, _D), lambda i: (0, 0)),
            pl.BlockSpec((_D, _D), lambda i: (0, 0)),
        ],
        out_specs=pl.BlockSpec((_BR, _D), lambda i: (i, 0)),
        out_shape=jax.ShapeDtypeStruct((_N, _D), jnp.float32),
    )(acc1, acc1, h1p, deg2, deg2, b1, W2)


def _tc_post_layer(acc2, h2p, deg2, b2):
    def body(aA_ref, aB_ref, h_ref, dA_ref, dB_ref, b_ref, o_ref):
        dis = _dis_from(dA_ref[...], dB_ref[...])
        pre = (aA_ref[0] + aB_ref[0] + h_ref[...]) * dis + b_ref[...]
        o_ref[...] = jnp.maximum(pre, 0.0)

    return pl.pallas_call(
        body,
        grid=(_NBLK,),
        in_specs=[
            pl.BlockSpec((1, _BR, _D), lambda i: (0, i, 0)),
            pl.BlockSpec((1, _BR, _D), lambda i: (1, i, 0)),
            pl.BlockSpec((_BR, _D), lambda i: (i, 0)),
            pl.BlockSpec((1, _BR, 1), lambda i: (0, i, 0)),
            pl.BlockSpec((1, _BR, 1), lambda i: (1, i, 0)),
            pl.BlockSpec((1, _D), lambda i: (0, 0)),
        ],
        out_specs=pl.BlockSpec((_BR, _D), lambda i: (i, 0)),
        out_shape=jax.ShapeDtypeStruct((_N, _D), jnp.float32),
    )(acc2, acc2, h2p, deg2, deg2, b2)


def _tc_head(acc2, h2p, deg2, b2, batch_col, f1W, f1b, f2W, f2b):
    def body(aA_ref, aB_ref, hp_ref, dA_ref, dB_ref, bb_ref,
             b_ref, w1_ref, b1_ref, w2_ref, b2_ref, o_ref, gf_ref):
        dis = _dis_from(dA_ref[...], dB_ref[...])
        h2 = (aA_ref[0] + aB_ref[0] + hp_ref[...]) * dis + bb_ref[...]
        h2 = jnp.maximum(h2, 0.0)
        h3 = h2.reshape(_N // 8, 8, _D)
        b3 = b_ref[...].reshape(_N // 8, 8, 1)
        ids0 = b3[:, 0, :]
        ids7 = b3[:, 7, :]
        low = jnp.max(jnp.where(b3 == ids0[:, None, :], h3, -jnp.inf), axis=1)
        high = jnp.max(jnp.where(b3 == ids7[:, None, :], h3, -jnp.inf), axis=1)

        def seg(g, carry):
            ml = jnp.max(jnp.where(ids0 == g, low, -jnp.inf),
                         axis=0, keepdims=True)
            mh = jnp.max(jnp.where(ids7 == g, high, -jnp.inf),
                         axis=0, keepdims=True)
            gf_ref[pl.ds(g, 1), :] = jnp.maximum(ml, mh)
            return carry

        lax.fori_loop(0, _G, seg, 0)

        @pl.when(jnp.max(ids7 - ids0) >= 2)
        def _():
            def seg_full(g, carry):
                m = jnp.where(b_ref[...] == g, h2, -jnp.inf)
                gf_ref[pl.ds(g, 1), :] = jnp.max(m, axis=0, keepdims=True)
                return carry

            lax.fori_loop(0, _G, seg_full, 0)
        gf = gf_ref[...]
        z1 = jnp.dot(gf, w1_ref[...], preferred_element_type=jnp.float32)
        z1 = jnp.maximum(z1 + b1_ref[...], 0.0)
        z2 = jnp.dot(z1, w2_ref[...], preferred_element_type=jnp.float32)
        z2 = z2 + b2_ref[...]
        zmax = jnp.max(z2, axis=1, keepdims=True)
        lse = jnp.log(jnp.sum(jnp.exp(z2 - zmax), axis=1, keepdims=True)) + zmax
        o_ref[...] = z2 - lse

    return pl.pallas_call(
        body,
        grid=(1,),
        in_specs=[
            pl.BlockSpec((1, _N, _D), lambda i: (0, 0, 0)),
            pl.BlockSpec((1, _N, _D), lambda i: (1, 0, 0)),
            pl.BlockSpec((_N, _D), lambda i: (0, 0)),
            pl.BlockSpec((1, _N, 1), lambda i: (0, 0, 0)),
            pl.BlockSpec((1, _N, 1), lambda i: (1, 0, 0)),
            pl.BlockSpec((1, _D), lambda i: (0, 0)),
            pl.BlockSpec((_N, 1), lambda i: (0, 0)),
            pl.BlockSpec((_D, _D), lambda i: (0, 0)),
            pl.BlockSpec((1, _D), lambda i: (0, 0)),
            pl.BlockSpec((_D, _D), lambda i: (0, 0)),
            pl.BlockSpec((1, _D), lambda i: (0, 0)),
        ],
        out_specs=pl.BlockSpec((_G, _D), lambda i: (0, 0)),
        scratch_shapes=[pltpu.VMEM((_G, _D), jnp.float32)],
        out_shape=jax.ShapeDtypeStruct((_G, _D), jnp.float32),
    )(acc2, acc2, h2p, deg2, deg2, b2, batch_col, f1W, f1b, f2W, f2b)


def kernel(x, edge_index, batch, W1, b1, W2, b2, fc1_W, fc1_b, fc2_W, fc2_b):
    src = edge_index[0].astype(jnp.int32)
    dst = edge_index[1].astype(jnp.int32)
    npad = _ES + _EF - _E
    src3 = jnp.concatenate(
        [src, jnp.zeros((npad,), jnp.int32)]).reshape(
            _NC * _NS * _HLF, _CHHX, _K)
    dst3 = jnp.concatenate(
        [dst, jnp.full((npad,), _N, jnp.int32)]).reshape(
            _NC * _NS * _HLF, _CHHX, _K)

    zero_hist = jnp.zeros((_HR, _K), jnp.float32)
    zero_rows = jnp.zeros((_RPS, _D), jnp.float32)
    iota_rows = jnp.arange(_HR, dtype=jnp.int32).reshape(1, _HR)

    deg2 = _sc_degree(dst3, iota_rows, zero_hist).reshape(_NC, _HR * _K, 1)

    h1p = _tc_scale_matmul(x, W1, deg2)
    acc1 = _sc_aggregate(h1p, src3, dst3, zero_rows).reshape(_NC, _NPAD, _D)
    h2p = _tc_mid_layer(acc1, h1p, deg2, b1.reshape(1, _D), W2)
    acc2 = _sc_aggregate(h2p, src3, dst3, zero_rows).reshape(_NC, _NPAD, _D)

    f1W = jnp.pad(fc1_W, ((0, 0), (0, _D - fc1_W.shape[1])))
    f1b = jnp.pad(fc1_b, (0, _D - fc1_b.shape[0])).reshape(1, _D)
    f2W = jnp.pad(fc2_W, ((0, _D - fc2_W.shape[0]), (0, _D - fc2_W.shape[1])))
    f2b = jnp.pad(fc2_b, (0, _D - fc2_b.shape[0]),
                  constant_values=-1e30).reshape(1, _D)

    out = _tc_head(acc2, h2p, deg2, b2.reshape(1, _D),
                   batch.astype(jnp.int32).reshape(_N, 1),
                   f1W, f1b, f2W, f2b)
    return out[:, :_C]


# speedup vs baseline: 1.0183x; 1.0132x over previous
import dataclasses
import functools

import jax
import jax.numpy as jnp
from jax import lax
from jax.experimental import pallas as pl
from jax.experimental.pallas import tpu as pltpu
from jax.experimental.pallas import tpu_sc as plsc

_N = 10000
_E = 320000
_D = 128
_G = 64
_C = 8

_NC = 2
_NS = 16
_L = 16

_K = 128
_KG = 64
_SLOW = 0
_CHS = 80
_CHF = 80
_RINGG = 4
_ES = _NS * _CHS * _K
_EF = _NS * _CHF * _K
_NPAD = 10112
_RPS = _NPAD // _NS
_HLF = 2
_CHHX = _CHF // _HLF

_NBLK = 10
_BR = _N // _NBLK

_RING = 2


def _sc_mesh():
    return plsc.VectorSubcoreMesh(
        core_axis_name="c", subcore_axis_name="s",
        num_cores=_NC, num_subcores=_NS)


_HR = 80


def _sc_degree(dst3, iota_rows, zero_rows):
    @functools.partial(
        pl.kernel,
        out_type=jax.ShapeDtypeStruct((_NC * _HR, _K), jnp.float32),
        mesh=_sc_mesh(),
        compiler_params=dataclasses.replace(pltpu.CompilerParams(),
                                            needs_layout_passes=False),
        scratch_types=[
            pltpu.VMEM((_HLF * _CHHX, _K), jnp.int32),
            pltpu.VMEM((_HR, _K), jnp.float32),
            pltpu.VMEM((1, _HR), jnp.int32),
            pltpu.VMEM_SHARED((_HR, _K), jnp.float32),
            pltpu.SemaphoreType.DMA,
        ],
    )
    def deg_kernel(dst_hbm, iota_hbm, zero_hbm, out_hbm,
                   dstv, histv, iotav, accsh, sem):
        c = lax.axis_index("c")
        s = lax.axis_index("s")
        w = c * _NS + s
        pltpu.sync_copy(zero_hbm, histv)
        pltpu.sync_copy(iota_hbm, iotav)
        for half in range(_HLF):
            pltpu.sync_copy(dst_hbm.at[w * _HLF + half],
                            dstv.at[pl.ds(half * _CHHX, _CHHX)])

        @pl.when(s < _HR // 8)
        def _():
            pltpu.sync_copy(zero_hbm.at[pl.ds(s * 8, 8)],
                            accsh.at[pl.ds(s * 8, 8)])

        ones = jnp.ones((_L,), jnp.float32)

        @pl.loop(0, _HLF * _CHHX)
        def _(j):
            @pl.loop(0, _K // _L)
            def _(t):
                idx = dstv[j, pl.ds(t * _L, _L)]
                plsc.addupdate_scatter(histv, [idx >> 7, idx & 127], ones)

        plsc.subcore_barrier()
        pltpu.sync_copy(histv, accsh.at[iotav.at[0]], add=True)
        plsc.subcore_barrier()

        @pl.when(s < _HR // 8)
        def _():
            pltpu.sync_copy(accsh.at[pl.ds(s * 8, 8)],
                            out_hbm.at[pl.ds(c * _HR + s * 8, 8)])

    return deg_kernel(dst3, iota_rows, zero_rows)


def _sc_aggregate(h, src3, dst3, zero_rows):
    @functools.partial(
        pl.kernel,
        out_type=jax.ShapeDtypeStruct((_NC * _NPAD, _D), jnp.float32),
        mesh=_sc_mesh(),
        scratch_types=[
            pltpu.VMEM((_CHHX, _K), jnp.int32),
            pltpu.VMEM((_CHHX, _K), jnp.int32),
            pltpu.VMEM((_K, _D), jnp.float32),
            pltpu.VMEM((_K, _D), jnp.float32),
            pltpu.SemaphoreType.DMA,
            pltpu.SemaphoreType.DMA,
            pltpu.SemaphoreType.DMA,
            pltpu.SemaphoreType.DMA,
            pltpu.VMEM_SHARED((_NPAD, _D), jnp.float32),
        ],
    )
    def agg_kernel(h_hbm, src_hbm, dst_hbm, zero_hbm, out_hbm,
                   srcv, dstv, rb0, rb1, sm0, sm1, ss0, ss1, accsh):
        rowbufs = (rb0, rb1)
        gsems = (sm0, sm1)
        ssems = (ss0, ss1)
        c = lax.axis_index("c")
        s = lax.axis_index("s")
        w = c * _NS + s
        pltpu.sync_copy(zero_hbm, accsh.at[pl.ds(s * _RPS, _RPS)])
        plsc.subcore_barrier()

        def wait_g(r):
            pltpu.make_async_copy(h_hbm.at[srcv.at[0]], rowbufs[r],
                                  gsems[r]).wait()

        def wait_s(r):
            pltpu.make_async_copy(rowbufs[r], accsh.at[dstv.at[0]],
                                  ssems[r]).wait()

        def run_edges(n_half):
            for half in range(_HLF):
                pltpu.sync_copy(src_hbm.at[w * _HLF + half], srcv)
                pltpu.sync_copy(dst_hbm.at[w * _HLF + half], dstv)
                for r in range(_RING):
                    pltpu.async_copy(h_hbm.at[srcv.at[r]], rowbufs[r],
                                     gsems[r])

                @pl.loop(0, n_half - _RING, step=_RING)
                def _(j):
                    for r in range(_RING):
                        wait_g(r)
                        pltpu.async_copy(rowbufs[r], accsh.at[dstv.at[j + r]],
                                         ssems[r], add=True)
                    for r in range(_RING):
                        wait_s(r)
                        pltpu.async_copy(h_hbm.at[srcv.at[j + _RING + r]],
                                         rowbufs[r], gsems[r])

                for r in range(_RING):
                    wait_g(r)
                    pltpu.async_copy(rowbufs[r],
                                     accsh.at[dstv.at[n_half - _RING + r]],
                                     ssems[r], add=True)
                for r in range(_RING):
                    wait_s(r)

        run_edges(_CHS // _HLF)

        plsc.subcore_barrier()
        pltpu.sync_copy(accsh.at[pl.ds(s * _RPS, _RPS)],
                        out_hbm.at[pl.ds(c * _NPAD + s * _RPS, _RPS)])

    return agg_kernel(h, src3, dst3, zero_rows)


def _dis_from(degA, degB):
    deg = degA[0, :, :1] + degB[0, :, :1] + 1.0
    return lax.rsqrt(deg)


def _tc_scale_matmul(x, W1, deg2):
    def body(x_ref, w_ref, dA_ref, dB_ref, o_ref):
        dis = _dis_from(dA_ref[...], dB_ref[...])
        h = jnp.dot(x_ref[...], w_ref[...], preferred_element_type=jnp.float32)
        o_ref[...] = h * dis

    return pl.pallas_call(
        body,
        grid=(_NBLK,),
        in_specs=[
            pl.BlockSpec((_BR, _D), lambda i: (i, 0)),
            pl.BlockSpec((_D, _D), lambda i: (0, 0)),
            pl.BlockSpec((1, _BR, 1), lambda i: (0, i, 0)),
            pl.BlockSpec((1, _BR, 1), lambda i: (1, i, 0)),
        ],
        out_specs=pl.BlockSpec((_BR, _D), lambda i: (i, 0)),
        out_shape=jax.ShapeDtypeStruct((_N, _D), jnp.float32),
    )(x, W1, deg2, deg2)


def _tc_mid_layer(acc1, h1p, deg2, b1, W2):
    def body(aA_ref, aB_ref, h_ref, dA_ref, dB_ref, b_ref, w_ref, o_ref):
        dis = _dis_from(dA_ref[...], dB_ref[...])
        pre = (aA_ref[0] + aB_ref[0] + h_ref[...]) * dis + b_ref[...]
        x2 = jnp.maximum(pre, 0.0)
        o_ref[...] = jnp.dot(x2, w_ref[...],
                             preferred_element_type=jnp.float32) * dis

    return pl.pallas_call(
        body,
        grid=(_NBLK,),
        in_specs=[
            pl.BlockSpec((1, _BR, _D), lambda i: (0, i, 0)),
            pl.BlockSpec((1, _BR, _D), lambda i: (1, i, 0)),
            pl.BlockSpec((_BR, _D), lambda i: (i, 0)),
            pl.BlockSpec((1, _BR, 1), lambda i: (0, i, 0)),
            pl.BlockSpec((1, _BR, 1), lambda i: (1, i, 0)),
            pl.BlockSpec((1, _D), lambda i: (0, 0)),
            pl.BlockSpec((_D, _D), lambda i: (0, 0)),
        ],
        out_specs=pl.BlockSpec((_BR, _D), lambda i: (i, 0)),
        out_shape=jax.ShapeDtypeStruct((_N, _D), jnp.float32),
    )(acc1, acc1, h1p, deg2, deg2, b1, W2)


def _tc_post_layer(acc2, h2p, deg2, b2):
    def body(aA_ref, aB_ref, h_ref, dA_ref, dB_ref, b_ref, o_ref):
        dis = _dis_from(dA_ref[...], dB_ref[...])
        pre = (aA_ref[0] + aB_ref[0] + h_ref[...]) * dis + b_ref[...]
        o_ref[...] = jnp.maximum(pre, 0.0)

    return pl.pallas_call(
        body,
        grid=(_NBLK,),
        in_specs=[
            pl.BlockSpec((1, _BR, _D), lambda i: (0, i, 0)),
            pl.BlockSpec((1, _BR, _D), lambda i: (1, i, 0)),
            pl.BlockSpec((_BR, _D), lambda i: (i, 0)),
            pl.BlockSpec((1, _BR, 1), lambda i: (0, i, 0)),
            pl.BlockSpec((1, _BR, 1), lambda i: (1, i, 0)),
            pl.BlockSpec((1, _D), lambda i: (0, 0)),
        ],
        out_specs=pl.BlockSpec((_BR, _D), lambda i: (i, 0)),
        out_shape=jax.ShapeDtypeStruct((_N, _D), jnp.float32),
    )(acc2, acc2, h2p, deg2, deg2, b2)


def _tc_head(acc2, h2p, deg2, b2, batch_col, f1W, f1b, f2W, f2b):
    def body(aA_ref, aB_ref, hp_ref, dA_ref, dB_ref, bb_ref,
             b_ref, w1_ref, b1_ref, w2_ref, b2_ref, o_ref, h2s_ref, gf_ref):
        dis = _dis_from(dA_ref[...], dB_ref[...])
        h2 = (aA_ref[0] + aB_ref[0] + hp_ref[...]) * dis + bb_ref[...]
        h2s_ref[...] = jnp.maximum(h2, 0.0)

        def seg(g, carry):
            m = jnp.where(b_ref[...] == g, h2s_ref[...], -jnp.inf)
            gf_ref[pl.ds(g, 1), :] = jnp.max(m, axis=0, keepdims=True)
            return carry

        lax.fori_loop(0, _G, seg, 0)
        gf = gf_ref[...]
        z1 = jnp.dot(gf, w1_ref[...], preferred_element_type=jnp.float32)
        z1 = jnp.maximum(z1 + b1_ref[...], 0.0)
        z2 = jnp.dot(z1, w2_ref[...], preferred_element_type=jnp.float32)
        z2 = z2 + b2_ref[...]
        zmax = jnp.max(z2, axis=1, keepdims=True)
        lse = jnp.log(jnp.sum(jnp.exp(z2 - zmax), axis=1, keepdims=True)) + zmax
        o_ref[...] = z2 - lse

    return pl.pallas_call(
        body,
        grid=(1,),
        in_specs=[
            pl.BlockSpec((1, _N, _D), lambda i: (0, 0, 0)),
            pl.BlockSpec((1, _N, _D), lambda i: (1, 0, 0)),
            pl.BlockSpec((_N, _D), lambda i: (0, 0)),
            pl.BlockSpec((1, _N, 1), lambda i: (0, 0, 0)),
            pl.BlockSpec((1, _N, 1), lambda i: (1, 0, 0)),
            pl.BlockSpec((1, _D), lambda i: (0, 0)),
            pl.BlockSpec((_N, 1), lambda i: (0, 0)),
            pl.BlockSpec((_D, _D), lambda i: (0, 0)),
            pl.BlockSpec((1, _D), lambda i: (0, 0)),
            pl.BlockSpec((_D, _D), lambda i: (0, 0)),
            pl.BlockSpec((1, _D), lambda i: (0, 0)),
        ],
        out_specs=pl.BlockSpec((_G, _D), lambda i: (0, 0)),
        scratch_shapes=[pltpu.VMEM((_N, _D), jnp.float32),
                        pltpu.VMEM((_G, _D), jnp.float32)],
        out_shape=jax.ShapeDtypeStruct((_G, _D), jnp.float32),
    )(acc2, acc2, h2p, deg2, deg2, b2, batch_col, f1W, f1b, f2W, f2b)


def kernel(x, edge_index, batch, W1, b1, W2, b2, fc1_W, fc1_b, fc2_W, fc2_b):
    src = edge_index[0].astype(jnp.int32)
    dst = edge_index[1].astype(jnp.int32)
    npad = _ES + _EF - _E
    src3 = jnp.concatenate(
        [src, jnp.zeros((npad,), jnp.int32)]).reshape(
            _NC * _NS * _HLF, _CHHX, _K)
    dst3 = jnp.concatenate(
        [dst, jnp.full((npad,), _N, jnp.int32)]).reshape(
            _NC * _NS * _HLF, _CHHX, _K)

    zero_hist = jnp.zeros((_HR, _K), jnp.float32)
    zero_rows = jnp.zeros((_RPS, _D), jnp.float32)
    iota_rows = jnp.arange(_HR, dtype=jnp.int32).reshape(1, _HR)

    deg2 = _sc_degree(dst3, iota_rows, zero_hist).reshape(_NC, _HR * _K, 1)

    h1p = _tc_scale_matmul(x, W1, deg2)
    acc1 = _sc_aggregate(h1p, src3, dst3, zero_rows).reshape(_NC, _NPAD, _D)
    h2p = _tc_mid_layer(acc1, h1p, deg2, b1.reshape(1, _D), W2)
    acc2 = _sc_aggregate(h2p, src3, dst3, zero_rows).reshape(_NC, _NPAD, _D)

    f1W = jnp.pad(fc1_W, ((0, 0), (0, _D - fc1_W.shape[1])))
    f1b = jnp.pad(fc1_b, (0, _D - fc1_b.shape[0])).reshape(1, _D)
    f2W = jnp.pad(fc2_W, ((0, _D - fc2_W.shape[0]), (0, _D - fc2_W.shape[1])))
    f2b = jnp.pad(fc2_b, (0, _D - fc2_b.shape[0]),
                  constant_values=-1e30).reshape(1, _D)

    out = _tc_head(acc2, h2p, deg2, b2.reshape(1, _D),
                   batch.astype(jnp.int32).reshape(_N, 1),
                   f1W, f1b, f2W, f2b)
    return out[:, :_C]
